# baseline (speedup 1.0000x reference)
import functools

import jax
import jax.numpy as jnp
from jax import lax
from jax.experimental import pallas as pl
from jax.experimental.pallas import tpu as pltpu
from jax.experimental.pallas import tpu_sc as plsc

MAX_REL = 32
HIDDEN = 16
SEQ_LEN = 2048
NUM_BIAS = 2 * MAX_REL + 1
E_ROWS = 2 * SEQ_LEN - 1
TOP_FILL = SEQ_LEN - 1 - MAX_REL
MID_OFF = TOP_FILL
BOT_OFF = MID_OFF + NUM_BIAS

NUM_WORKERS = 32
ROWS_PER_W = SEQ_LEN // NUM_WORKERS


def _fill_rows(ref, row_vec, start, stop):
    def body(r, _):
        ref[r, :] = row_vec
        return 0

    lax.fori_loop(start, stop, body, 0)


def _sc_body(bias_hbm, out_hbm, e_v, sem):
    cid = lax.axis_index("c")
    sid = lax.axis_index("s")
    wid = sid * 2 + cid

    pltpu.sync_copy(bias_hbm, e_v.at[pl.ds(MID_OFF, NUM_BIAS), :])
    _fill_rows(e_v, e_v[MID_OFF, :], 0, TOP_FILL)
    _fill_rows(e_v, e_v[BOT_OFF - 1, :], BOT_OFF, E_ROWS)

    base = wid * ROWS_PER_W
    copies = []
    for r in range(ROWS_PER_W):
        i = base + r
        copies.append(
            pltpu.async_copy(
                e_v.at[pl.ds(SEQ_LEN - 1 - i, SEQ_LEN), :],
                out_hbm.at[i],
                sem,
            )
        )
    for c in copies:
        c.wait()


def kernel(relative_attention_bias, seq_length):
    del seq_length
    mesh = plsc.VectorSubcoreMesh(core_axis_name="c", subcore_axis_name="s")
    run = functools.partial(
        pl.kernel,
        mesh=mesh,
        out_type=jax.ShapeDtypeStruct((SEQ_LEN, SEQ_LEN, HIDDEN), jnp.float32),
        scratch_types=[
            pltpu.VMEM((E_ROWS, HIDDEN), jnp.float32),
            pltpu.SemaphoreType.DMA,
        ],
        compiler_params=pltpu.CompilerParams(use_tc_tiling_on_sc=False),
    )(_sc_body)
    return run(relative_attention_bias.astype(jnp.float32))


# speedup vs baseline: 12.9152x; 12.9152x over previous
import functools

import jax
import jax.numpy as jnp
from jax import lax
from jax.experimental import pallas as pl
from jax.experimental.pallas import tpu as pltpu
from jax.experimental.pallas import tpu_sc as plsc

MAX_REL = 32
HIDDEN = 16
SEQ_LEN = 2048
NUM_BIAS = 2 * MAX_REL + 1
S1 = SEQ_LEN - 1
LO_END = S1 - MAX_REL
HI_BEG = S1 + MAX_REL

NUM_WORKERS = 32
PHASES = 4
W_ET = 1920 + SEQ_LEN
BAND_LO = 1984
BAND_W = 128


def _sc_body(bias_hbm, out_hbm, bias_v, band_v, etp_v, sem):
    cid = lax.axis_index("c")
    sid = lax.axis_index("s")
    wid = sid * 2 + cid

    pltpu.sync_copy(bias_hbm, bias_v)

    lane = jnp.arange(HIDDEN, dtype=jnp.int32)
    lane_band = lane * BAND_W
    row_lo = bias_v[pl.ds(0, HIDDEN)]
    row_hi = bias_v[pl.ds((NUM_BIAS - 1) * HIDDEN, HIDDEN)]
    for b in range(BAND_W):
        r = min(max(b - 31, 0), NUM_BIAS - 1)
        if r == 0:
            x = row_lo
        elif r == NUM_BIAS - 1:
            x = row_hi
        else:
            x = bias_v[pl.ds(r * HIDDEN, HIDDEN)]
        plsc.store_scatter(band_v, [lane_band + b], x)

    for p in range(PHASES):
        rho = wid * PHASES + p
        cb0 = ((LO_END - 15 - rho) // 16 + 1) * 16
        cb1 = ((HI_BEG + 15 - rho) // 16) * 16
        for h in range(HIDDEN):
            splat_lo = jnp.broadcast_to(row_lo[h], (HIDDEN,))
            splat_hi = jnp.broadcast_to(row_hi[h], (HIDDEN,))

            @plsc.parallel_loop(0, cb0, 16, unroll=8)
            def _(c, _h=h, _v=splat_lo):
                etp_v[_h, pl.ds(c, 16)] = _v

            @plsc.parallel_loop(cb0, cb1, 16)
            def _(c, _h=h):
                etp_v[_h, pl.ds(c, 16)] = band_v[pl.ds(_h * BAND_W + rho + c - BAND_LO, 16)]

            @plsc.parallel_loop(cb1, W_ET, 16, unroll=8)
            def _(c, _h=h, _v=splat_hi):
                etp_v[_h, pl.ds(c, 16)] = _v

        copies = []
        for t in range(SEQ_LEN // BAND_W):
            w = rho + BAND_W * t
            i = S1 - w
            copies.append(
                pltpu.async_copy(
                    etp_v.at[:, pl.ds(BAND_W * t, SEQ_LEN)], out_hbm.at[i], sem
                )
            )
        for c_ in copies:
            c_.wait()


def kernel(relative_attention_bias, seq_length):
    del seq_length
    mesh = plsc.VectorSubcoreMesh(core_axis_name="c", subcore_axis_name="s")
    run = functools.partial(
        pl.kernel,
        mesh=mesh,
        out_type=jax.ShapeDtypeStruct((SEQ_LEN, HIDDEN, SEQ_LEN), jnp.float32),
        scratch_types=[
            pltpu.VMEM((NUM_BIAS * HIDDEN,), jnp.float32),
            pltpu.VMEM((HIDDEN * BAND_W,), jnp.float32),
            pltpu.VMEM((HIDDEN, W_ET), jnp.float32),
            pltpu.SemaphoreType.DMA,
        ],
        compiler_params=pltpu.CompilerParams(
            use_tc_tiling_on_sc=True, needs_layout_passes=False
        ),
    )(_sc_body)
    outT = run(relative_attention_bias.astype(jnp.float32).reshape(-1))
    return jnp.swapaxes(outT, 1, 2)


# speedup vs baseline: 14.3126x; 1.1082x over previous
import functools

import jax
import jax.numpy as jnp
from jax import lax
from jax.experimental import pallas as pl
from jax.experimental.pallas import tpu as pltpu
from jax.experimental.pallas import tpu_sc as plsc

MAX_REL = 32
HIDDEN = 16
SEQ_LEN = 2048
NUM_BIAS = 2 * MAX_REL + 1
S1 = SEQ_LEN - 1
LO_END = S1 - MAX_REL
HI_BEG = S1 + MAX_REL

NUM_WORKERS = 32
PHASES = 4
W_ET = 1920 + SEQ_LEN
BAND_LO = 1984
BAND_W = 128


def _sc_body(bias_hbm, out_hbm, bias_v, band_v, etp0_v, etp1_v, sem0, sem1):
    cid = lax.axis_index("c")
    sid = lax.axis_index("s")
    wid = sid * 2 + cid

    pltpu.sync_copy(bias_hbm, bias_v)

    lane = jnp.arange(HIDDEN, dtype=jnp.int32)
    lane_band = lane * BAND_W
    row_lo = bias_v[pl.ds(0, HIDDEN)]
    row_hi = bias_v[pl.ds((NUM_BIAS - 1) * HIDDEN, HIDDEN)]
    for b in range(BAND_W):
        r = min(max(b - 31, 0), NUM_BIAS - 1)
        if r == 0:
            x = row_lo
        elif r == NUM_BIAS - 1:
            x = row_hi
        else:
            x = bias_v[pl.ds(r * HIDDEN, HIDDEN)]
        plsc.store_scatter(band_v, [lane_band + b], x)

    bufs = (etp0_v, etp1_v)
    sems = (sem0, sem1)
    inflight = [None, None]
    for p in range(PHASES):
        rho = wid * PHASES + p
        etp_v = bufs[p % 2]
        sem = sems[p % 2]
        cb0 = ((LO_END - 15 - rho) // 16 + 1) * 16
        cb1 = ((HI_BEG + 15 - rho) // 16) * 16
        if inflight[p % 2] is not None:
            for c_ in inflight[p % 2]:
                c_.wait()
        if p < 2:
            for h in range(HIDDEN):
                splat_lo = jnp.broadcast_to(row_lo[h], (HIDDEN,))
                splat_hi = jnp.broadcast_to(row_hi[h], (HIDDEN,))

                @plsc.parallel_loop(0, cb0, 16, unroll=8)
                def _(c, _h=h, _v=splat_lo, _e=etp_v):
                    _e[_h, pl.ds(c, 16)] = _v

                @plsc.parallel_loop(cb0, cb1, 16)
                def _(c, _h=h, _e=etp_v, _r=rho):
                    _e[_h, pl.ds(c, 16)] = band_v[pl.ds(_h * BAND_W + _r + c - BAND_LO, 16)]

                @plsc.parallel_loop(cb1, W_ET, 16, unroll=8)
                def _(c, _h=h, _v=splat_hi, _e=etp_v):
                    _e[_h, pl.ds(c, 16)] = _v
        else:
            cb1_old = ((HI_BEG + 15 - (rho - 2)) // 16) * 16
            for h in range(HIDDEN):

                @plsc.parallel_loop(cb0, cb1_old, 16)
                def _(c, _h=h, _e=etp_v, _r=rho):
                    _e[_h, pl.ds(c, 16)] = band_v[pl.ds(_h * BAND_W + _r + c - BAND_LO, 16)]

        copies = []
        for t in range(SEQ_LEN // BAND_W):
            w = rho + BAND_W * t
            i = S1 - w
            copies.append(
                pltpu.async_copy(
                    etp_v.at[:, pl.ds(BAND_W * t, SEQ_LEN)], out_hbm.at[i], sem
                )
            )
        inflight[p % 2] = copies
    for cs in inflight:
        if cs is not None:
            for c_ in cs:
                c_.wait()


def kernel(relative_attention_bias, seq_length):
    del seq_length
    mesh = plsc.VectorSubcoreMesh(core_axis_name="c", subcore_axis_name="s")
    run = functools.partial(
        pl.kernel,
        mesh=mesh,
        out_type=jax.ShapeDtypeStruct((SEQ_LEN, HIDDEN, SEQ_LEN), jnp.float32),
        scratch_types=[
            pltpu.VMEM((NUM_BIAS * HIDDEN,), jnp.float32),
            pltpu.VMEM((HIDDEN * BAND_W,), jnp.float32),
            pltpu.VMEM((HIDDEN, W_ET), jnp.float32),
            pltpu.VMEM((HIDDEN, W_ET), jnp.float32),
            pltpu.SemaphoreType.DMA,
            pltpu.SemaphoreType.DMA,
        ],
        compiler_params=pltpu.CompilerParams(
            use_tc_tiling_on_sc=True, needs_layout_passes=False
        ),
    )(_sc_body)
    outT = run(relative_attention_bias.astype(jnp.float32).reshape(-1))
    return jnp.swapaxes(outT, 1, 2)
